# baseline (speedup 1.0000x reference)
import functools

import jax
import jax.numpy as jnp
from jax.experimental import pallas as pl


ALPHA = 33
C = ALPHA + 1


def _onehot_body(tok_ref, eng_ref, out_ref):
    tok = tok_ref[...]
    eng = eng_ref[...]
    r = tok.shape[0]
    iota = jax.lax.broadcasted_iota(jnp.int32, (r, C), 1)
    onehot = iota == tok
    motif = (iota == ALPHA) & (eng <= -1.0)
    out_ref[...] = (onehot | motif).astype(jnp.float32)


@functools.partial(jax.jit, static_argnames=("block_rows",))
def _run(tokens, energy_scores, block_rows=2048):
    nb, nt = tokens.shape
    n = nb * nt
    tok2 = tokens.reshape(n, 1)
    eng2 = energy_scores.reshape(n, 1)
    out2 = pl.pallas_call(
        _onehot_body,
        grid=(n // block_rows,),
        in_specs=[
            pl.BlockSpec((block_rows, 1), lambda i: (i, 0)),
            pl.BlockSpec((block_rows, 1), lambda i: (i, 0)),
        ],
        out_specs=pl.BlockSpec((block_rows, C), lambda i: (i, 0)),
        out_shape=jax.ShapeDtypeStruct((n, C), jnp.float32),
    )(tok2, eng2)
    return out2.reshape(nb, nt, C)


def kernel(tokens, energy_scores):
    return _run(tokens, energy_scores)


# speedup vs baseline: 1.3710x; 1.3710x over previous
import functools

import jax
import jax.numpy as jnp
from jax.experimental import pallas as pl
from jax.experimental.pallas import tpu as pltpu


ALPHA = 33
C = ALPHA + 1
P = 64
L = P * C


def _body(tok_ref, eng_ref, out_ref, s_tok_ref, s_m_ref, c_ref):
    @pl.when(pl.program_id(0) == 0)
    def _init():
        l_iota = jax.lax.broadcasted_iota(jnp.int32, (P, L), 1)
        p_iota = jax.lax.broadcasted_iota(jnp.int32, (P, L), 0)
        d = l_iota - C * p_iota
        in_group = (d >= 0) & (d < C)
        s_tok_ref[...] = ((d >= 0) & (d < ALPHA)).astype(jnp.float32)
        s_m_ref[...] = (d == ALPHA).astype(jnp.float32)
        c_ref[...] = jnp.sum(
            jnp.where(in_group, d, 0), axis=0, keepdims=True
        ).astype(jnp.float32)

    tokf = tok_ref[...].astype(jnp.float32)
    m = jnp.where(eng_ref[...] <= -1.0, float(ALPHA), -1.0)
    key = (
        jnp.dot(tokf, s_tok_ref[...], preferred_element_type=jnp.float32)
        + jnp.dot(m, s_m_ref[...], preferred_element_type=jnp.float32)
    )
    out_ref[...] = (key == c_ref[...]).astype(jnp.float32)


@functools.partial(jax.jit, static_argnames=("block_rows",))
def _run(tokens, energy_scores, block_rows=256):
    nb, nt = tokens.shape
    n = nb * nt
    rows = n // P
    tok2 = tokens.reshape(rows, P)
    eng2 = energy_scores.reshape(rows, P)
    out2 = pl.pallas_call(
        _body,
        grid=(rows // block_rows,),
        in_specs=[
            pl.BlockSpec((block_rows, P), lambda i: (i, 0)),
            pl.BlockSpec((block_rows, P), lambda i: (i, 0)),
        ],
        out_specs=pl.BlockSpec((block_rows, L), lambda i: (i, 0)),
        out_shape=jax.ShapeDtypeStruct((rows, L), jnp.float32),
        scratch_shapes=[
            pltpu.VMEM((P, L), jnp.float32),
            pltpu.VMEM((P, L), jnp.float32),
            pltpu.VMEM((1, L), jnp.float32),
        ],
    )(tok2, eng2)
    return out2.reshape(nb, nt, C)


def kernel(tokens, energy_scores):
    return _run(tokens, energy_scores)


# speedup vs baseline: 2.1962x; 1.6018x over previous
import functools

import jax
import jax.numpy as jnp
from jax.experimental import pallas as pl
from jax.experimental.pallas import tpu as pltpu


ALPHA = 33
C = ALPHA + 1
P = 128
L = P * C


def _body(tok_ref, eng_ref, out_ref, s_tok_ref, s_m_ref, c_ref):
    @pl.when((pl.program_id(0) == 0) & (pl.program_id(1) == 0))
    def _init():
        l_iota = jax.lax.broadcasted_iota(jnp.int32, (P, L), 1)
        p_iota = jax.lax.broadcasted_iota(jnp.int32, (P, L), 0)
        d = l_iota - C * p_iota
        in_group = (d >= 0) & (d < C)
        s_tok_ref[...] = ((d >= 0) & (d < ALPHA)).astype(jnp.float32)
        s_m_ref[...] = (d == ALPHA).astype(jnp.float32)
        c_ref[...] = jnp.sum(
            jnp.where(in_group, d, 0), axis=0, keepdims=True
        ).astype(jnp.float32)

    tokf = tok_ref[...].astype(jnp.float32)
    m = jnp.where(eng_ref[...] <= -1.0, float(ALPHA), -1.0)
    key = (
        jnp.dot(tokf, s_tok_ref[...], preferred_element_type=jnp.float32)
        + jnp.dot(m, s_m_ref[...], preferred_element_type=jnp.float32)
    )
    out_ref[...] = (key == c_ref[...]).astype(jnp.float32)


@functools.partial(jax.jit, static_argnames=("block_rows",))
def _run(tokens, energy_scores, block_rows=32):
    nb, nt = tokens.shape
    out2 = pl.pallas_call(
        _body,
        grid=(nb // block_rows, nt // P),
        in_specs=[
            pl.BlockSpec((block_rows, P), lambda i, j: (i, j)),
            pl.BlockSpec((block_rows, P), lambda i, j: (i, j)),
        ],
        out_specs=pl.BlockSpec((block_rows, L), lambda i, j: (i, j)),
        out_shape=jax.ShapeDtypeStruct((nb, nt * C), jnp.float32),
        scratch_shapes=[
            pltpu.VMEM((P, L), jnp.float32),
            pltpu.VMEM((P, L), jnp.float32),
            pltpu.VMEM((1, L), jnp.float32),
        ],
    )(tokens, energy_scores)
    return out2.reshape(nb, nt, C)


def kernel(tokens, energy_scores):
    return _run(tokens, energy_scores)


# speedup vs baseline: 20.5657x; 9.3644x over previous
import functools

import jax
import jax.numpy as jnp
from jax.experimental import pallas as pl


ALPHA = 33
C = ALPHA + 1


def _planes_body(tok_ref, eng_ref, out_ref):
    tok = tok_ref[...]
    for c in range(ALPHA):
        out_ref[c] = (tok == c).astype(jnp.float32)
    out_ref[ALPHA] = (eng_ref[...] <= -1.0).astype(jnp.float32)


@functools.partial(jax.jit, static_argnames=("block_rows",))
def _run(tokens, energy_scores, block_rows=8):
    nb, nt = tokens.shape
    outp = pl.pallas_call(
        _planes_body,
        grid=(nb // block_rows,),
        in_specs=[
            pl.BlockSpec((block_rows, nt), lambda i: (i, 0)),
            pl.BlockSpec((block_rows, nt), lambda i: (i, 0)),
        ],
        out_specs=pl.BlockSpec((C, block_rows, nt), lambda i: (0, i, 0)),
        out_shape=jax.ShapeDtypeStruct((C, nb, nt), jnp.float32),
    )(tokens, energy_scores)
    return jnp.transpose(outp, (1, 2, 0))


def kernel(tokens, energy_scores):
    return _run(tokens, energy_scores)


# speedup vs baseline: 25.7205x; 1.2507x over previous
import functools

import jax
import jax.numpy as jnp
from jax.experimental import pallas as pl


ALPHA = 33
C = ALPHA + 1


def _planes_body(tok_ref, eng_ref, out_ref):
    tok = tok_ref[...]
    for c in range(ALPHA):
        out_ref[c] = (tok == c).astype(jnp.float32)
    out_ref[ALPHA] = (eng_ref[...] <= -1.0).astype(jnp.float32)


@functools.partial(jax.jit, static_argnames=("block_rows",))
def _run(tokens, energy_scores, block_rows=16):
    nb, nt = tokens.shape
    outp = pl.pallas_call(
        _planes_body,
        grid=(nb // block_rows,),
        in_specs=[
            pl.BlockSpec((block_rows, nt), lambda i: (i, 0)),
            pl.BlockSpec((block_rows, nt), lambda i: (i, 0)),
        ],
        out_specs=pl.BlockSpec((C, block_rows, nt), lambda i: (0, i, 0)),
        out_shape=jax.ShapeDtypeStruct((C, nb, nt), jnp.float32),
    )(tokens, energy_scores)
    return jnp.transpose(outp, (1, 2, 0))


def kernel(tokens, energy_scores):
    return _run(tokens, energy_scores)
